# Initial kernel scaffold; baseline (speedup 1.0000x reference)
import jax
import jax.numpy as jnp
from jax.experimental import pallas as pl


def kernel(scores, particle_number, edge_index):
    raise NotImplementedError("write your pallas kernel here")


# speedup vs baseline: 394.4880x; 394.4880x over previous
import functools

import jax
import jax.numpy as jnp
from jax import lax
from jax.experimental import pallas as pl
from jax.experimental.pallas import tpu as pltpu
from jax.experimental.pallas import tpu_sc as plsc

N_NODES = 100000
N_EDGES = 6400000
NC = 2
NS = 16
L = 16

PER_TILE = N_EDGES // NS
CHUNK = 3200
N_CHUNKS = PER_TILE // CHUNK


def _body(scores_hbm, pn_hbm, edges_hbm, score_out, mask_out,
          table_v, idx_a, idx_b, out_v):
    c = lax.axis_index("c")
    s = lax.axis_index("s")
    is_score = c == 0

    @pl.when(is_score)
    def _():
        pltpu.sync_copy(scores_hbm, table_v)

        def chunk(k, carry):
            base = s * PER_TILE + k * CHUNK
            pltpu.sync_copy(edges_hbm.at[pl.ds(base, CHUNK)], idx_a)

            def grp(i, carry2):
                sl = pl.ds(i * L, L)
                out_v[sl] = plsc.load_gather(table_v, [idx_a[sl]])
                return carry2

            lax.fori_loop(0, CHUNK // L, grp, 0)
            pltpu.sync_copy(out_v, score_out.at[pl.ds(base, CHUNK)])
            return carry

        lax.fori_loop(0, N_CHUNKS, chunk, 0)

    @pl.when(jnp.logical_not(is_score))
    def _():
        pltpu.sync_copy(pn_hbm, table_v)

        def chunk(k, carry):
            base = s * PER_TILE + k * CHUNK
            pltpu.sync_copy(edges_hbm.at[pl.ds(base, CHUNK)], idx_a)
            pltpu.sync_copy(edges_hbm.at[pl.ds(N_EDGES + base, CHUNK)], idx_b)

            def grp(i, carry2):
                sl = pl.ds(i * L, L)
                va = plsc.load_gather(table_v, [idx_a[sl]])
                vb = plsc.load_gather(table_v, [idx_b[sl]])
                out_v[sl] = (va == vb).astype(jnp.int32)
                return carry2

            lax.fori_loop(0, CHUNK // L, grp, 0)
            pltpu.sync_copy(out_v, mask_out.at[pl.ds(base, CHUNK)])
            return carry

        lax.fori_loop(0, N_CHUNKS, chunk, 0)


_sc_call = pl.kernel(
    _body,
    out_type=[
        jax.ShapeDtypeStruct((N_EDGES,), jnp.int32),
        jax.ShapeDtypeStruct((N_EDGES,), jnp.int32),
    ],
    mesh=plsc.VectorSubcoreMesh(core_axis_name="c", subcore_axis_name="s",
                                num_cores=NC, num_subcores=NS),
    scratch_types=[
        pltpu.VMEM((N_NODES,), jnp.int32),
        pltpu.VMEM((CHUNK,), jnp.int32),
        pltpu.VMEM((CHUNK,), jnp.int32),
        pltpu.VMEM((CHUNK,), jnp.int32),
    ],
    compiler_params=pltpu.CompilerParams(needs_layout_passes=False),
)


def kernel(scores, particle_number, edge_index):
    scores_i32 = lax.bitcast_convert_type(scores.reshape(-1), jnp.int32)
    pn_i32 = particle_number.astype(jnp.int32)
    score_bits, mask = _sc_call(scores_i32, pn_i32, edge_index.reshape(-1))
    score_neigh = lax.bitcast_convert_type(score_bits, jnp.float32)
    return score_neigh, mask.astype(bool)


# speedup vs baseline: 490.1025x; 1.2424x over previous
import functools

import jax
import jax.numpy as jnp
from jax import lax
from jax.experimental import pallas as pl
from jax.experimental.pallas import tpu as pltpu
from jax.experimental.pallas import tpu_sc as plsc

N_NODES = 100000
N_EDGES = 6400000
NC = 2
NS = 16
L = 16

PER_TILE = N_EDGES // NS
CHUNK = 3200
N_CHUNKS = PER_TILE // CHUNK


def _body(scores_hbm, pn_hbm, edges_hbm, score_out, mask_out,
          table_v, idx_a, idx_b, out_v):
    c = lax.axis_index("c")
    s = lax.axis_index("s")
    is_score = c == 0

    @pl.when(is_score)
    def _():
        pltpu.sync_copy(scores_hbm, table_v)

        def chunk(k, carry):
            base = s * PER_TILE + k * CHUNK
            pltpu.sync_copy(edges_hbm.at[pl.ds(base, CHUNK)], idx_a)

            @plsc.parallel_loop(0, CHUNK, step=L, unroll=8)
            def _grp(off):
                sl = pl.ds(off, L)
                out_v[sl] = plsc.load_gather(table_v, [idx_a[sl]])

            pltpu.sync_copy(out_v, score_out.at[pl.ds(base, CHUNK)])
            return carry

        lax.fori_loop(0, N_CHUNKS, chunk, 0)

    @pl.when(jnp.logical_not(is_score))
    def _():
        pltpu.sync_copy(pn_hbm, table_v)

        def chunk(k, carry):
            base = s * PER_TILE + k * CHUNK
            pltpu.sync_copy(edges_hbm.at[pl.ds(base, CHUNK)], idx_a)
            pltpu.sync_copy(edges_hbm.at[pl.ds(N_EDGES + base, CHUNK)], idx_b)

            @plsc.parallel_loop(0, CHUNK, step=L, unroll=8)
            def _grp(off):
                sl = pl.ds(off, L)
                va = plsc.load_gather(table_v, [idx_a[sl]])
                vb = plsc.load_gather(table_v, [idx_b[sl]])
                out_v[sl] = (va == vb).astype(jnp.int32)

            pltpu.sync_copy(out_v, mask_out.at[pl.ds(base, CHUNK)])
            return carry

        lax.fori_loop(0, N_CHUNKS, chunk, 0)


_sc_call = pl.kernel(
    _body,
    out_type=[
        jax.ShapeDtypeStruct((N_EDGES,), jnp.int32),
        jax.ShapeDtypeStruct((N_EDGES,), jnp.int32),
    ],
    mesh=plsc.VectorSubcoreMesh(core_axis_name="c", subcore_axis_name="s",
                                num_cores=NC, num_subcores=NS),
    scratch_types=[
        pltpu.VMEM((N_NODES,), jnp.int32),
        pltpu.VMEM((CHUNK,), jnp.int32),
        pltpu.VMEM((CHUNK,), jnp.int32),
        pltpu.VMEM((CHUNK,), jnp.int32),
    ],
    compiler_params=pltpu.CompilerParams(needs_layout_passes=False),
)


def kernel(scores, particle_number, edge_index):
    scores_i32 = lax.bitcast_convert_type(scores.reshape(-1), jnp.int32)
    pn_i32 = particle_number.astype(jnp.int32)
    score_bits, mask = _sc_call(scores_i32, pn_i32, edge_index.reshape(-1))
    score_neigh = lax.bitcast_convert_type(score_bits, jnp.float32)
    return score_neigh, mask.astype(bool)
